# baseline (speedup 1.0000x reference)
import functools

import jax
import jax.numpy as jnp
from jax import lax
from jax.experimental import pallas as pl
from jax.experimental.pallas import tpu as pltpu
from jax.experimental.pallas import tpu_sc as plsc

_NC = 2
_NS = 16
_NW = _NC * _NS

_NBUF = 8


def _emb_call(b0, b1, d):
    mesh = plsc.VectorSubcoreMesh(core_axis_name="c", subcore_axis_name="s")
    rows_per_w = b0 // _NW

    @functools.partial(
        pl.kernel,
        mesh=mesh,
        out_type=jax.ShapeDtypeStruct((b0, b1, d), jnp.float32),
        compiler_params=pltpu.CompilerParams(use_tc_tiling_on_sc=False),
        scratch_types=[
            pltpu.VMEM((rows_per_w, 64), jnp.int32),
            pltpu.VMEM((_NBUF, 56, d), jnp.float32),
        ]
        + [pltpu.SemaphoreType.DMA] * (2 * _NBUF),
    )
    def emb(idx_hbm, table_hbm, out_hbm, idx_v, rows_v, *sems):
        gsems = sems[:_NBUF]
        osems = sems[_NBUF:]
        wid = lax.axis_index("s") * _NC + lax.axis_index("c")
        base = wid * rows_per_w
        pltpu.sync_copy(idx_hbm.at[pl.ds(base, rows_per_w)], idx_v)

        def fire_gather(b, p):
            pltpu.async_copy(
                table_hbm.at[idx_v.at[p, pl.ds(0, 56)]], rows_v.at[b], gsems[b]
            )

        def wait_gather(b):
            pltpu.make_async_copy(
                table_hbm.at[idx_v.at[0, pl.ds(0, 56)]], rows_v.at[b], gsems[b]
            ).wait()

        def fire_out(b, p):
            pltpu.async_copy(
                rows_v.at[b, pl.ds(0, b1)], out_hbm.at[base + p], osems[b]
            )

        def wait_out(b):
            pltpu.make_async_copy(
                rows_v.at[b, pl.ds(0, b1)], out_hbm.at[base], osems[b]
            ).wait()

        for b in range(_NBUF):
            fire_gather(b, b)

        def body(step, carry):
            i = step * _NBUF
            for b in range(_NBUF):
                wait_gather(b)
                fire_out(b, i + b)
            for b in range(_NBUF):
                wait_out(b)
                fire_gather(b, i + b + _NBUF)
            return carry

        lax.fori_loop(0, (rows_per_w - _NBUF) // _NBUF, body, 0)

        for b in range(_NBUF):
            wait_gather(b)
            fire_out(b, rows_per_w - _NBUF + b)
        for b in range(_NBUF):
            wait_out(b)

    return emb


def kernel(city, table):
    b0, b1 = city.shape
    v, d = table.shape
    idx = jnp.pad(city.astype(jnp.int32), ((0, 0), (0, 64 - b1)))
    return _emb_call(b0, b1, d)(idx, table)


# speedup vs baseline: 3.2260x; 3.2260x over previous
import functools

import jax
import jax.numpy as jnp
from jax import lax
from jax.experimental import pallas as pl
from jax.experimental.pallas import tpu as pltpu
from jax.experimental.pallas import tpu_sc as plsc

_NC = 2
_NS = 16
_NW = _NC * _NS

_CHUNK = 256
_NBUF = 5


def _emb_call(n_idx, d, n_ch):
    mesh = plsc.VectorSubcoreMesh(core_axis_name="c", subcore_axis_name="s")
    b_per_w = n_ch * _CHUNK

    @functools.partial(
        pl.kernel,
        mesh=mesh,
        out_type=jax.ShapeDtypeStruct((n_idx, d), jnp.float32),
        compiler_params=pltpu.CompilerParams(use_tc_tiling_on_sc=False),
        scratch_types=[
            pltpu.VMEM((n_ch, _CHUNK), jnp.int32),
            pltpu.VMEM((_NBUF, _CHUNK, d), jnp.float32),
        ]
        + [pltpu.SemaphoreType.DMA] * _NBUF,
    )
    def emb(idx_hbm, table_hbm, out_hbm, idx_v, rows_v, *gsems):
        wid = lax.axis_index("s") * _NC + lax.axis_index("c")
        base = wid * b_per_w
        pltpu.sync_copy(idx_hbm.at[wid], idx_v)

        for b in range(_NBUF):
            pltpu.async_copy(table_hbm.at[idx_v.at[b]], rows_v.at[b], gsems[b])

        def body(step, carry):
            i = step * _NBUF
            for b in range(_NBUF):
                g = i + b
                pltpu.make_async_copy(
                    table_hbm.at[idx_v.at[b]], rows_v.at[b], gsems[b]
                ).wait()
                pltpu.sync_copy(
                    rows_v.at[b], out_hbm.at[pl.ds(base + g * _CHUNK, _CHUNK)]
                )
                pltpu.async_copy(
                    table_hbm.at[idx_v.at[g + _NBUF]], rows_v.at[b], gsems[b]
                )
            return carry

        lax.fori_loop(0, (n_ch - _NBUF) // _NBUF, body, 0)

        for b in range(_NBUF):
            g = n_ch - _NBUF + b
            pltpu.make_async_copy(
                table_hbm.at[idx_v.at[b]], rows_v.at[b], gsems[b]
            ).wait()
            pltpu.sync_copy(
                rows_v.at[b], out_hbm.at[pl.ds(base + g * _CHUNK, _CHUNK)]
            )

    return emb


def kernel(city, table):
    b0, b1 = city.shape
    v, d = table.shape
    n_idx = b0 * b1
    n_ch = n_idx // (_NW * _CHUNK)
    idx = city.reshape(_NW, n_ch, _CHUNK).astype(jnp.int32)
    out = _emb_call(n_idx, d, n_ch)(idx, table)
    return out.reshape(b0, b1, d)
